# Initial kernel scaffold; baseline (speedup 1.0000x reference)
import jax
import jax.numpy as jnp
from jax.experimental import pallas as pl


def kernel(lattice_values, neighbor_indices, weight, bias):
    raise NotImplementedError("write your pallas kernel here")


# speedup vs baseline: 6.5542x; 6.5542x over previous
import functools

import jax
import jax.numpy as jnp
from jax import lax
from jax.experimental import pallas as pl
from jax.experimental.pallas import tpu as pltpu
from jax.experimental.pallas import tpu_sc as plsc

N = 100000
VAL_DIM = 128
FE = 9
NF = 32

NW = 32
CHUNK = 128
CHUNKS_PER_W = 25
NPAD = NW * CHUNKS_PER_W * CHUNK
CF = CHUNK * FE
VREGS = CF // 16


def _proj_body(x_ref, w_ref, o_ref):
    o_ref[...] = jnp.dot(x_ref[...], w_ref[...],
                         preferred_element_type=jnp.float32)


def _project(lattice_values, w_r):
    blk = 2000
    return pl.pallas_call(
        _proj_body,
        grid=(N // blk,),
        in_specs=[
            pl.BlockSpec((blk, VAL_DIM), lambda i: (i, 0)),
            pl.BlockSpec((VAL_DIM, FE * NF), lambda i: (0, 0)),
        ],
        out_specs=pl.BlockSpec((blk, FE * NF), lambda i: (i, 0)),
        out_shape=jax.ShapeDtypeStruct((N, FE * NF), jnp.float32),
    )(lattice_values, w_r)


@functools.partial(
    pl.kernel,
    mesh=plsc.VectorSubcoreMesh(core_axis_name="c", subcore_axis_name="s"),
    compiler_params=pltpu.CompilerParams(use_tc_tiling_on_sc=False),
    out_type=jax.ShapeDtypeStruct((NPAD, NF), jnp.float32),
    scratch_types=[
        pltpu.VMEM((CF,), jnp.int32),
        pltpu.VMEM((CF,), jnp.int32),
        pltpu.VMEM((CF,), jnp.int32),
        pltpu.VMEM((CF, NF), jnp.float32),
        pltpu.VMEM((CHUNK, NF), jnp.float32),
        pltpu.VMEM((NF,), jnp.float32),
        pltpu.SemaphoreType.DMA,
    ],
)
def _sc_gather(table, nbr, bias, out, idx_v, flat_v, kpat_v, rows_v,
               acc_v, bias_v, sem):
    cid = lax.axis_index("c")
    sid = lax.axis_index("s")
    wid = sid * 2 + cid
    iota16 = lax.iota(jnp.int32, 16)

    fe_c = jnp.full((16,), FE, dtype=jnp.int32)
    for v in range(VREGS):
        m0 = v * 16
        mvec = iota16 + jnp.int32(m0)
        kpat_v[pl.ds(m0, 16)] = lax.rem(mvec, fe_c)

    pltpu.sync_copy(bias, bias_v)
    blo = bias_v[pl.ds(0, 16)]
    bhi = bias_v[pl.ds(16, 16)]

    def chunk_body(t, carry):
        g = wid * CHUNKS_PER_W + t
        pltpu.sync_copy(nbr.at[pl.ds(g * CF, CF)], idx_v)
        for v in range(VREGS):
            m0 = v * 16
            flat_v[pl.ds(m0, 16)] = (idx_v[pl.ds(m0, 16)] * fe_c
                                     + kpat_v[pl.ds(m0, 16)])
        pltpu.async_copy(table.at[flat_v], rows_v, sem).wait()

        def vert_body(j, c):
            base = j * FE
            lo = blo
            hi = bhi
            for k in range(FE):
                lo = lo + rows_v[base + k, pl.ds(0, 16)]
                hi = hi + rows_v[base + k, pl.ds(16, 16)]
            acc_v[j, pl.ds(0, 16)] = lo
            acc_v[j, pl.ds(16, 16)] = hi
            return c

        lax.fori_loop(0, CHUNK, vert_body, 0)
        pltpu.sync_copy(acc_v, out.at[pl.ds(g * CHUNK, CHUNK)])
        return carry

    lax.fori_loop(0, CHUNKS_PER_W, chunk_body, 0)


def kernel(lattice_values, neighbor_indices, weight, bias):
    w_r = weight.reshape(FE, VAL_DIM, NF).transpose(1, 0, 2).reshape(
        VAL_DIM, FE * NF)
    proj = _project(lattice_values, w_r)
    table = proj.reshape(N * FE, NF)
    nbr_flat = neighbor_indices.astype(jnp.int32).reshape(-1)
    nbr_pad = jnp.pad(nbr_flat, (0, NPAD * FE - N * FE))
    out = _sc_gather(table, nbr_pad, bias)
    return out[:N]


# speedup vs baseline: 8.2506x; 1.2588x over previous
import functools

import jax
import jax.numpy as jnp
from jax import lax
from jax.experimental import pallas as pl
from jax.experimental.pallas import tpu as pltpu
from jax.experimental.pallas import tpu_sc as plsc

N = 100000
VAL_DIM = 128
FE = 9
NF = 32

NW = 32
CHUNK = 128
CHUNKS_PER_W = 25
NPAD = NW * CHUNKS_PER_W * CHUNK
CF = CHUNK * FE
VREGS = CF // 16


def _proj_body(x_ref, w_ref, o_ref):
    o_ref[0] = jnp.dot(x_ref[...], w_ref[0],
                       preferred_element_type=jnp.float32)


def _project(lattice_values, w3):
    blk = 4000
    return pl.pallas_call(
        _proj_body,
        grid=(N // blk, 3),
        in_specs=[
            pl.BlockSpec((blk, VAL_DIM), lambda i, t: (i, 0)),
            pl.BlockSpec((1, VAL_DIM, VAL_DIM), lambda i, t: (t, 0, 0)),
        ],
        out_specs=pl.BlockSpec((1, blk, VAL_DIM), lambda i, t: (t, i, 0)),
        out_shape=jax.ShapeDtypeStruct((3, N, VAL_DIM), jnp.float32),
    )(lattice_values, w3)


@functools.partial(
    pl.kernel,
    mesh=plsc.VectorSubcoreMesh(core_axis_name="c", subcore_axis_name="s"),
    compiler_params=pltpu.CompilerParams(use_tc_tiling_on_sc=False),
    out_type=jax.ShapeDtypeStruct((NPAD, NF), jnp.float32),
    scratch_types=[
        pltpu.VMEM((CF,), jnp.int32),
        pltpu.VMEM((CF,), jnp.int32),
        pltpu.VMEM((CF,), jnp.int32),
        pltpu.VMEM((CF, NF), jnp.float32),
        pltpu.VMEM((CHUNK, NF), jnp.float32),
        pltpu.VMEM((NF,), jnp.float32),
        pltpu.SemaphoreType.DMA,
    ],
)
def _sc_gather(table, nbr, bias, out, idx_v, flat_v, kpat_v, rows_v,
               acc_v, bias_v, sem):
    cid = lax.axis_index("c")
    sid = lax.axis_index("s")
    wid = sid * 2 + cid
    iota16 = lax.iota(jnp.int32, 16)

    fe_c = jnp.full((16,), FE, dtype=jnp.int32)
    four_c = jnp.full((16,), 4, dtype=jnp.int32)
    eight_c = jnp.full((16,), 8, dtype=jnp.int32)
    offb_c = jnp.full((16,), 4 * N - 4, dtype=jnp.int32)
    offc_c = jnp.full((16,), 8 * N - 8, dtype=jnp.int32)
    for v in range(VREGS):
        m0 = v * 16
        mvec = iota16 + jnp.int32(m0)
        k = lax.rem(mvec, fe_c)
        kpat_v[pl.ds(m0, 16)] = k + jnp.where(
            k < four_c, 0, jnp.where(k < eight_c, offb_c, offc_c))

    pltpu.sync_copy(bias, bias_v)
    blo = bias_v[pl.ds(0, 16)]
    bhi = bias_v[pl.ds(16, 16)]

    def chunk_body(t, carry):
        g = wid * CHUNKS_PER_W + t
        pltpu.sync_copy(nbr.at[pl.ds(g * CF, CF)], idx_v)
        for v in range(VREGS):
            m0 = v * 16
            flat_v[pl.ds(m0, 16)] = (idx_v[pl.ds(m0, 16)] * four_c
                                     + kpat_v[pl.ds(m0, 16)])
        pltpu.async_copy(table.at[flat_v], rows_v, sem).wait()

        def vert_body(j, c):
            base = j * FE
            lo = blo
            hi = bhi
            for k in range(FE):
                lo = lo + rows_v[base + k, pl.ds(0, 16)]
                hi = hi + rows_v[base + k, pl.ds(16, 16)]
            acc_v[j, pl.ds(0, 16)] = lo
            acc_v[j, pl.ds(16, 16)] = hi
            return c

        lax.fori_loop(0, CHUNK, vert_body, 0)
        pltpu.sync_copy(acc_v, out.at[pl.ds(g * CHUNK, CHUNK)])
        return carry

    lax.fori_loop(0, CHUNKS_PER_W, chunk_body, 0)


def kernel(lattice_values, neighbor_indices, weight, bias):
    w_r = weight.reshape(FE, VAL_DIM, NF).transpose(1, 0, 2).reshape(
        VAL_DIM, FE * NF)
    w3 = jnp.pad(w_r, ((0, 0), (0, 3 * VAL_DIM - FE * NF))).reshape(
        VAL_DIM, 3, VAL_DIM).transpose(1, 0, 2)
    proj = _project(lattice_values, w3)
    table = proj.reshape(3 * N * VAL_DIM // NF, NF)
    nbr_flat = neighbor_indices.astype(jnp.int32).reshape(-1)
    nbr_pad = jnp.pad(nbr_flat, (0, NPAD * FE - N * FE))
    out = _sc_gather(table, nbr_pad, bias)
    return out[:N]


# speedup vs baseline: 20.2316x; 2.4521x over previous
import functools

import jax
import jax.numpy as jnp
from jax import lax
from jax.experimental import pallas as pl
from jax.experimental.pallas import tpu as pltpu
from jax.experimental.pallas import tpu_sc as plsc

N = 100000
VAL_DIM = 128
FE = 9
NF = 32

CHUNK = 128
NCHUNK = N // CHUNK
TAIL = N - NCHUNK * CHUNK
CF = CHUNK * FE
VREGS = CF // 16
TROWS = TAIL * FE


def _slot_off(k):
    return (k // 4) * 4 * N + (k % 4) if k < 8 else 8 * N


def _proj_body(x_ref, w_ref, o_ref):
    x = x_ref[...]
    for t in range(3):
        o_ref[t] = jnp.dot(x, w_ref[t], preferred_element_type=jnp.float32)


def _project(lattice_values, w3):
    blk = 4000
    return pl.pallas_call(
        _proj_body,
        grid=(N // blk,),
        in_specs=[
            pl.BlockSpec((blk, VAL_DIM), lambda i: (i, 0)),
            pl.BlockSpec((3, VAL_DIM, VAL_DIM), lambda i: (0, 0, 0)),
        ],
        out_specs=pl.BlockSpec((3, blk, VAL_DIM), lambda i: (0, i, 0)),
        out_shape=jax.ShapeDtypeStruct((3, N, VAL_DIM), jnp.float32),
    )(lattice_values, w3)


@functools.partial(
    pl.kernel,
    mesh=plsc.VectorSubcoreMesh(core_axis_name="c", subcore_axis_name="s"),
    compiler_params=pltpu.CompilerParams(use_tc_tiling_on_sc=False),
    out_type=jax.ShapeDtypeStruct((N, NF), jnp.float32),
    scratch_types=[
        pltpu.VMEM((2, FE, CHUNK), jnp.int32),
        pltpu.VMEM((2, CF), jnp.int32),
        pltpu.VMEM((2, CF, NF), jnp.float32),
        pltpu.VMEM((2, CHUNK, NF), jnp.float32),
        pltpu.VMEM((NF,), jnp.float32),
        pltpu.SemaphoreType.DMA((2,)),
        pltpu.SemaphoreType.DMA((2,)),
    ],
)
def _sc_gather(table, nbr_t, bias, out, idx_v, flat_v, rows_v, acc_v,
               bias_v, sem_g, sem_o):
    cid = lax.axis_index("c")
    sid = lax.axis_index("s")
    wid = sid * 2 + cid
    cbase = 24 * wid + jnp.minimum(wid, 13)
    ncch = jnp.where(wid < 13, 25, 24)

    four_c = jnp.full((16,), 4, dtype=jnp.int32)
    pltpu.sync_copy(bias, bias_v)
    blo = bias_v[pl.ds(0, 16)]
    bhi = bias_v[pl.ds(16, 16)]

    def issue(t):
        b = lax.rem(t, 2)
        g = cbase + t
        pltpu.sync_copy(nbr_t.at[:, pl.ds(g * CHUNK, CHUNK)], idx_v.at[b])
        for v in range(VREGS):
            k = v // 8
            j0 = (v % 8) * 16
            flat_v[b, pl.ds(v * 16, 16)] = (
                idx_v[b, k, pl.ds(j0, 16)] * four_c
                + jnp.full((16,), _slot_off(k), dtype=jnp.int32))
        pltpu.async_copy(table.at[flat_v.at[b]], rows_v.at[b], sem_g.at[b])

    issue(0)

    def chunk_body(t, carry):
        b = lax.rem(t, 2)
        g = cbase + t

        @pl.when(t + 1 < ncch)
        def _():
            issue(t + 1)

        pltpu.make_async_copy(table.at[flat_v.at[b]], rows_v.at[b],
                              sem_g.at[b]).wait()

        @pl.when(t >= 2)
        def _():
            gp = g - 2
            pltpu.make_async_copy(
                acc_v.at[b], out.at[pl.ds(gp * CHUNK, CHUNK)],
                sem_o.at[b]).wait()

        def vert_body(j, c):
            lo = blo
            hi = bhi
            for k in range(FE):
                lo = lo + rows_v[b, k * CHUNK + j, pl.ds(0, 16)]
                hi = hi + rows_v[b, k * CHUNK + j, pl.ds(16, 16)]
            acc_v[b, j, pl.ds(0, 16)] = lo
            acc_v[b, j, pl.ds(16, 16)] = hi
            return c

        lax.fori_loop(0, CHUNK, vert_body, 0)
        pltpu.async_copy(acc_v.at[b], out.at[pl.ds(g * CHUNK, CHUNK)],
                         sem_o.at[b])
        return carry

    lax.fori_loop(0, ncch, chunk_body, 0)

    for dt in (2, 1):
        t = ncch - dt
        b = lax.rem(t, 2)
        pltpu.make_async_copy(
            acc_v.at[b], out.at[pl.ds((cbase + t) * CHUNK, CHUNK)],
            sem_o.at[b]).wait()

    @pl.when(wid == 31)
    def _():
        pltpu.sync_copy(nbr_t.at[:, pl.ds(NCHUNK * CHUNK, TAIL)],
                        idx_v.at[0, :, pl.ds(0, TAIL)])
        for v in range(TROWS // 16):
            k = v // 2
            j0 = (v % 2) * 16
            flat_v[0, pl.ds(v * 16, 16)] = (
                idx_v[0, k, pl.ds(j0, 16)] * four_c
                + jnp.full((16,), _slot_off(k), dtype=jnp.int32))
        pltpu.async_copy(table.at[flat_v.at[0, pl.ds(0, TROWS)]],
                         rows_v.at[0, pl.ds(0, TROWS)], sem_g.at[0]).wait()

        def tail_body(j, c):
            lo = blo
            hi = bhi
            for k in range(FE):
                lo = lo + rows_v[0, k * TAIL + j, pl.ds(0, 16)]
                hi = hi + rows_v[0, k * TAIL + j, pl.ds(16, 16)]
            acc_v[0, j, pl.ds(0, 16)] = lo
            acc_v[0, j, pl.ds(16, 16)] = hi
            return c

        lax.fori_loop(0, TAIL, tail_body, 0)
        pltpu.sync_copy(acc_v.at[0, pl.ds(0, TAIL)],
                        out.at[pl.ds(NCHUNK * CHUNK, TAIL)])


def kernel(lattice_values, neighbor_indices, weight, bias):
    w_r = weight.reshape(FE, VAL_DIM, NF).transpose(1, 0, 2).reshape(
        VAL_DIM, FE * NF)
    w3 = jnp.pad(w_r, ((0, 0), (0, 3 * VAL_DIM - FE * NF))).reshape(
        VAL_DIM, 3, VAL_DIM).transpose(1, 0, 2)
    proj = _project(lattice_values, w3)
    table = proj.reshape(3 * N * VAL_DIM // NF, NF)
    nbr_t = neighbor_indices.astype(jnp.int32).T
    return _sc_gather(table, nbr_t, bias)
